# baseline (speedup 1.0000x reference)
import functools

import jax
import jax.numpy as jnp
from jax import lax
from jax.experimental import pallas as pl
from jax.experimental.pallas import tpu as pltpu
from jax.experimental.pallas import tpu_sc as plsc

ROWS = 64
N = 32768
LANES = 16
CHUNK = N // LANES
NSEG = 4
SEG = CHUNK // NSEG
RADIX_BITS = 8
NBINS = 1 << RADIX_BITS
NPASS = 32 // RADIX_BITS
NUM_CORES = 2
NUM_SUBCORES = 16
NW = NUM_CORES * NUM_SUBCORES
ROWS_PER_W = ROWS // NW
PITCH = CHUNK + 1
NSKEW = LANES * PITCH


def _build_sorter():
    mesh = plsc.VectorSubcoreMesh(core_axis_name="c", subcore_axis_name="s")

    @functools.partial(
        pl.kernel,
        mesh=mesh,
        compiler_params=pltpu.CompilerParams(needs_layout_passes=False),
        out_type=(
            jax.ShapeDtypeStruct((ROWS, N), jnp.float32),
            jax.ShapeDtypeStruct((ROWS, N), jnp.int32),
        ),
        scratch_types=[
            pltpu.VMEM((NSKEW,), jnp.float32),
            pltpu.VMEM((NSKEW,), jnp.int32),
            pltpu.VMEM((NSKEW,), jnp.float32),
            pltpu.VMEM((NBINS * LANES,), jnp.int32),
            pltpu.VMEM((NBINS * LANES,), jnp.int32),
            pltpu.VMEM((NBINS * LANES,), jnp.int32),
            pltpu.VMEM((NBINS * LANES,), jnp.int32),
        ],
    )
    def body(x_hbm, vals_hbm, idx_hbm, K, A, B, H0, H1, H2, H3):
        HT = (H0, H1, H2, H3)
        wid = lax.axis_index("s") * NUM_CORES + lax.axis_index("c")
        lane = lax.iota(jnp.int32, LANES)
        lane_base = lane * CHUNK
        lane_base_sk = lane * PITCH
        ones = jnp.ones((LANES,), jnp.int32)
        top_bit = jnp.full((LANES,), jnp.int32(-2147483648))
        v31 = jnp.full((LANES,), 31, jnp.int32)
        v4 = jnp.full((LANES,), 4, jnp.int32)
        v11 = jnp.full((LANES,), 11, jnp.int32)
        v15 = jnp.full((LANES,), 15, jnp.int32)
        v23 = jnp.full((LANES,), 23, jnp.int32)
        vdmask = jnp.full((LANES,), NBINS - 1, jnp.int32)
        vsmask = jnp.full((LANES,), (NBINS - 1) << 4, jnp.int32)
        vidxmask = jnp.full((LANES,), (1 << 15) - 1, jnp.int32)

        def key_at(idxv):
            sk = idxv + lax.shift_right_logical(idxv, v11)
            return plsc.bitcast(plsc.load_gather(K, [sk]), jnp.int32)

        def slot_from_digit(d):
            return lax.bitwise_or(lax.shift_left(d, v4), lane)

        for r in range(ROWS_PER_W):
            row = wid * ROWS_PER_W + r
            pltpu.sync_copy(x_hbm.at[row], B.at[pl.ds(0, N)])

            @plsc.parallel_loop(0, CHUNK, unroll=4)
            def _(t):
                kv = plsc.bitcast(B[pl.ds(t * LANES, LANES)], jnp.int32)
                sgn = lax.shift_right_arithmetic(kv, v31)
                fl = lax.bitwise_xor(kv, lax.bitwise_or(sgn, top_bit))
                K[pl.ds(t * LANES + (t >> 7), LANES)] = plsc.bitcast(fl, jnp.float32)

            for p in range(NPASS):

                def load_aug(pos):
                    return plsc.load_gather(A, [pos])

                def store_aug(pos, augv):
                    plsc.store_scatter(B, [pos], plsc.bitcast(augv, jnp.float32))

                def load_aug2(pos):
                    return plsc.bitcast(plsc.load_gather(B, [pos]), jnp.int32)

                def store_out(rank, payload):
                    if p + 1 < NPASS:
                        rank = rank + lax.shift_right_logical(rank, v11)
                    plsc.store_scatter(A, [rank], payload)

                @plsc.parallel_loop(0, NBINS, unroll=8)
                def _(d):
                    z = jnp.zeros((LANES,), jnp.int32)
                    for s in range(NSEG):
                        HT[s][pl.ds(d * LANES, LANES)] = z

                def count_walk(t, c):
                    poss = [lane_base_sk + (s * SEG + t) for s in range(NSEG)]
                    if p == 0:
                        idxs = [lane_base + (s * SEG + t) for s in range(NSEG)]
                        keys = [key_at(iv) for iv in idxs]
                        d0s = [lax.bitwise_and(k, vdmask) for k in keys]
                        slots = [slot_from_digit(d) for d in d0s]
                        augs = [lax.bitwise_or(idxs[s], lax.shift_left(d0s[s], v15))
                                for s in range(NSEG)]
                    else:
                        augs = [load_aug(ps) for ps in poss]
                        slots = [lax.bitwise_or(
                            lax.bitwise_and(
                                lax.shift_right_logical(a, v11), vsmask), lane)
                            for a in augs]
                    cnts = [plsc.load_gather(HT[s], [slots[s]])
                            for s in range(NSEG)]
                    for s in range(NSEG):
                        plsc.store_scatter(HT[s], [slots[s]], cnts[s] + ones)
                    for s in range(NSEG):
                        store_aug(poss[s], lax.bitwise_or(
                            augs[s], lax.shift_left(cnts[s], v23)))
                    return c
                lax.fori_loop(0, SEG, count_walk, 0, unroll=2)

                def scan(d, run):
                    hs = [HT[s][pl.ds(d * LANES, LANES)] for s in range(NSEG)]
                    tot = hs[0] + hs[1] + hs[2] + hs[3]
                    excl = plsc.cumsum(tot) - tot
                    b = excl + run
                    for s in range(NSEG):
                        HT[s][pl.ds(d * LANES, LANES)] = b
                        if s + 1 < NSEG:
                            b = b + hs[s]
                    return run + jnp.sum(tot)
                lax.fori_loop(0, NBINS, scan, jnp.int32(0), unroll=2)

                @plsc.parallel_loop(0, SEG, unroll=4)
                def _(t):
                    for s in range(NSEG):
                        pos = lane_base_sk + (s * SEG + t)
                        augv = load_aug2(pos)
                        idxv = lax.bitwise_and(augv, vidxmask)
                        slot = lax.bitwise_or(
                            lax.bitwise_and(
                                lax.shift_right_logical(augv, v11), vsmask), lane)
                        cnt = lax.shift_right_logical(augv, v23)
                        rank = plsc.load_gather(HT[s], [slot]) + cnt
                        if p + 1 < NPASS:
                            dnext = lax.bitwise_and(
                                lax.shift_right_logical(
                                    key_at(idxv),
                                    jnp.full((LANES,), (p + 1) * RADIX_BITS, jnp.int32)),
                                vdmask)
                            payload = lax.bitwise_or(
                                idxv, lax.shift_left(dnext, v15))
                        else:
                            payload = idxv
                        store_out(rank, payload)

            @plsc.parallel_loop(0, CHUNK, unroll=4)
            def _(t):
                iv = A[pl.ds(t * LANES, LANES)]
                fv = key_at(iv)
                sgn = lax.shift_right_arithmetic(lax.bitwise_not(fv), v31)
                kv = lax.bitwise_xor(fv, lax.bitwise_or(sgn, top_bit))
                B[pl.ds(t * LANES, LANES)] = plsc.bitcast(kv, jnp.float32)

            pltpu.sync_copy(B.at[pl.ds(0, N)], vals_hbm.at[row])
            pltpu.sync_copy(A.at[pl.ds(0, N)], idx_hbm.at[row])

    return body


def kernel(input_tensor, output_tensor, indice_tensor):
    del output_tensor, indice_tensor
    return _build_sorter()(input_tensor)


# speedup vs baseline: 1.0854x; 1.0854x over previous
import functools

import jax
import jax.numpy as jnp
from jax import lax
from jax.experimental import pallas as pl
from jax.experimental.pallas import tpu as pltpu
from jax.experimental.pallas import tpu_sc as plsc

ROWS = 64
N = 32768
LANES = 16
CHUNK = N // LANES
NSEG = 4
SEG = CHUNK // NSEG
RADIX_BITS = 8
NBINS = 1 << RADIX_BITS
NPASS = 32 // RADIX_BITS
NUM_CORES = 2
NUM_SUBCORES = 16
NW = NUM_CORES * NUM_SUBCORES
ROWS_PER_W = ROWS // NW
PITCH = CHUNK + 1
NSKEW = LANES * PITCH


def _build_sorter():
    mesh = plsc.VectorSubcoreMesh(core_axis_name="c", subcore_axis_name="s")

    @functools.partial(
        pl.kernel,
        mesh=mesh,
        compiler_params=pltpu.CompilerParams(needs_layout_passes=False),
        out_type=(
            jax.ShapeDtypeStruct((ROWS, N), jnp.float32),
            jax.ShapeDtypeStruct((ROWS, N), jnp.int32),
        ),
        scratch_types=[
            pltpu.VMEM((NSKEW,), jnp.float32),
            pltpu.VMEM((NSKEW,), jnp.int32),
            pltpu.VMEM((NSKEW,), jnp.float32),
            pltpu.VMEM((NBINS * LANES,), jnp.int32),
            pltpu.VMEM((NBINS * LANES,), jnp.int32),
            pltpu.VMEM((NBINS * LANES,), jnp.int32),
            pltpu.VMEM((NBINS * LANES,), jnp.int32),
        ],
    )
    def body(x_hbm, vals_hbm, idx_hbm, K, A, B, H0, H1, H2, H3):
        HT = (H0, H1, H2, H3)
        wid = lax.axis_index("s") * NUM_CORES + lax.axis_index("c")
        lane = lax.iota(jnp.int32, LANES)
        lane_base = lane * CHUNK
        lane_base_sk = lane * PITCH
        ones = jnp.ones((LANES,), jnp.int32)
        top_bit = jnp.full((LANES,), jnp.int32(-2147483648))
        v31 = jnp.full((LANES,), 31, jnp.int32)
        v4 = jnp.full((LANES,), 4, jnp.int32)
        v11 = jnp.full((LANES,), 11, jnp.int32)
        v15 = jnp.full((LANES,), 15, jnp.int32)
        v23 = jnp.full((LANES,), 23, jnp.int32)
        vdmask = jnp.full((LANES,), NBINS - 1, jnp.int32)
        vsmask = jnp.full((LANES,), (NBINS - 1) << 4, jnp.int32)
        vidxmask = jnp.full((LANES,), (1 << 15) - 1, jnp.int32)

        def key_at(idxv):
            sk = idxv + lax.shift_right_logical(idxv, v11)
            return plsc.bitcast(plsc.load_gather(K, [sk]), jnp.int32)

        def slot_from_digit(d):
            return lax.bitwise_or(lax.shift_left(d, v4), lane)

        for r in range(ROWS_PER_W):
            row = wid * ROWS_PER_W + r
            pltpu.sync_copy(x_hbm.at[row], B.at[pl.ds(0, N)])

            @plsc.parallel_loop(0, CHUNK, unroll=4)
            def _(t):
                kv = plsc.bitcast(B[pl.ds(t * LANES, LANES)], jnp.int32)
                sgn = lax.shift_right_arithmetic(kv, v31)
                fl = lax.bitwise_xor(kv, lax.bitwise_or(sgn, top_bit))
                K[pl.ds(t * LANES + (t >> 7), LANES)] = plsc.bitcast(fl, jnp.float32)

            for p in range(NPASS):

                def load_aug(pos):
                    return plsc.load_gather(A, [pos])

                def store_aug(pos, augv):
                    plsc.store_scatter(B, [pos], plsc.bitcast(augv, jnp.float32))

                def load_aug2(pos):
                    return plsc.bitcast(plsc.load_gather(B, [pos]), jnp.int32)

                def store_out(rank, payload):
                    if p + 1 < NPASS:
                        rank = rank + lax.shift_right_logical(rank, v11)
                    plsc.store_scatter(A, [rank], payload)

                @plsc.parallel_loop(0, NBINS, unroll=8)
                def _(d):
                    z = jnp.zeros((LANES,), jnp.int32)
                    for s in range(NSEG):
                        HT[s][pl.ds(d * LANES, LANES)] = z

                U = 2
                def count_walk(t, c):
                    t0 = t * U
                    poss = [[lane_base_sk + (s * SEG + t0 + j) for j in range(U)]
                            for s in range(NSEG)]
                    if p == 0:
                        idxs = [[lane_base + (s * SEG + t0 + j) for j in range(U)]
                                for s in range(NSEG)]
                        keys = [[key_at(iv) for iv in row] for row in idxs]
                        d0s = [[lax.bitwise_and(k, vdmask) for k in row]
                               for row in keys]
                        slots = [[slot_from_digit(d) for d in row] for row in d0s]
                        augs = [[lax.bitwise_or(idxs[s][j],
                                                lax.shift_left(d0s[s][j], v15))
                                 for j in range(U)] for s in range(NSEG)]
                    else:
                        augs = [[load_aug(ps) for ps in row] for row in poss]
                        slots = [[lax.bitwise_or(
                            lax.bitwise_and(
                                lax.shift_right_logical(a, v11), vsmask), lane)
                            for a in row] for row in augs]
                    cnts = [[plsc.load_gather(HT[s], [slots[s][j]])
                             for j in range(U)] for s in range(NSEG)]
                    for s in range(NSEG):
                        for j in range(1, U):
                            for i in range(j):
                                eq = jnp.where(
                                    slots[s][i] == slots[s][j], ones,
                                    jnp.zeros((LANES,), jnp.int32))
                                cnts[s][j] = cnts[s][j] + eq
                    for s in range(NSEG):
                        for j in range(U):
                            plsc.store_scatter(HT[s], [slots[s][j]],
                                               cnts[s][j] + ones)
                    for s in range(NSEG):
                        for j in range(U):
                            store_aug(poss[s][j], lax.bitwise_or(
                                augs[s][j], lax.shift_left(cnts[s][j], v23)))
                    return c
                lax.fori_loop(0, SEG // U, count_walk, 0, unroll=1)

                def scan(d, run):
                    hs = [HT[s][pl.ds(d * LANES, LANES)] for s in range(NSEG)]
                    tot = hs[0] + hs[1] + hs[2] + hs[3]
                    excl = plsc.cumsum(tot) - tot
                    b = excl + run
                    for s in range(NSEG):
                        HT[s][pl.ds(d * LANES, LANES)] = b
                        if s + 1 < NSEG:
                            b = b + hs[s]
                    return run + jnp.sum(tot)
                lax.fori_loop(0, NBINS, scan, jnp.int32(0), unroll=2)

                @plsc.parallel_loop(0, SEG, unroll=4)
                def _(t):
                    for s in range(NSEG):
                        pos = lane_base_sk + (s * SEG + t)
                        augv = load_aug2(pos)
                        idxv = lax.bitwise_and(augv, vidxmask)
                        slot = lax.bitwise_or(
                            lax.bitwise_and(
                                lax.shift_right_logical(augv, v11), vsmask), lane)
                        cnt = lax.shift_right_logical(augv, v23)
                        rank = plsc.load_gather(HT[s], [slot]) + cnt
                        if p + 1 < NPASS:
                            dnext = lax.bitwise_and(
                                lax.shift_right_logical(
                                    key_at(idxv),
                                    jnp.full((LANES,), (p + 1) * RADIX_BITS, jnp.int32)),
                                vdmask)
                            payload = lax.bitwise_or(
                                idxv, lax.shift_left(dnext, v15))
                        else:
                            payload = idxv
                        store_out(rank, payload)

            @plsc.parallel_loop(0, CHUNK, unroll=4)
            def _(t):
                iv = A[pl.ds(t * LANES, LANES)]
                fv = key_at(iv)
                sgn = lax.shift_right_arithmetic(lax.bitwise_not(fv), v31)
                kv = lax.bitwise_xor(fv, lax.bitwise_or(sgn, top_bit))
                B[pl.ds(t * LANES, LANES)] = plsc.bitcast(kv, jnp.float32)

            pltpu.sync_copy(B.at[pl.ds(0, N)], vals_hbm.at[row])
            pltpu.sync_copy(A.at[pl.ds(0, N)], idx_hbm.at[row])

    return body


def kernel(input_tensor, output_tensor, indice_tensor):
    del output_tensor, indice_tensor
    return _build_sorter()(input_tensor)
